# baseline (speedup 1.0000x reference)
import functools

import jax
import jax.numpy as jnp
from jax.experimental import pallas as pl
from jax.experimental.pallas import tpu as pltpu


def _fused_kernel(xm_ref, xh_ref, w_ref, b_ref, o_ref, *, th, w, cin, cgrp):
    lanes = (th + 1) * w
    xt = jnp.concatenate([xm_ref[...], xh_ref[...]], axis=-1)

    col = jax.lax.broadcasted_iota(jnp.int32, (cin, lanes), 1) % w
    zero = jnp.zeros((), xt.dtype)
    taps = []
    for dy in range(3):
        for dx in range(3):
            start = dy * w + dx - 1
            if start >= 0:
                t = xt[:, start:start + lanes]
            else:
                t = jnp.concatenate([xt[:, :1], xt[:, :lanes - 1]], axis=1)
            if dx == 0:
                t = jnp.where(col == 0, zero, t)
            elif dx == 2:
                t = jnp.where(col == w - 1, zero, t)
            taps.append(t)
    rhs = jnp.concatenate(taps, axis=0)

    acc = jnp.dot(w_ref[...], rhs,
                  preferred_element_type=jnp.float32) + b_ref[...]

    n_out = th * w
    cur = [acc[p * cgrp:(p + 1) * cgrp, w:] for p in range(4)]
    up2 = acc[2 * cgrp:3 * cgrp, :n_out]
    up3 = acc[3 * cgrp:4 * cgrp, :n_out]

    lane = jax.lax.broadcasted_iota(jnp.int32, (cgrp, n_out), 1)
    col0 = (lane % w) == 0
    row0 = jnp.logical_and(pl.program_id(1) == 0, lane < w)

    lcur1 = jnp.where(col0, cur[0], pltpu.roll(cur[1], 1, axis=1))
    lcur3 = jnp.where(col0, cur[2], pltpu.roll(cur[3], 1, axis=1))
    up2f = jnp.where(row0, cur[0], up2)
    up3f = jnp.where(row0, cur[1], up3)
    lup3 = jnp.where(row0, lcur1,
                     jnp.where(col0, up2, pltpu.roll(up3, 1, axis=1)))

    o00 = (cur[0] + lcur1) + (up2f + lup3)
    o01 = (cur[0] + cur[1]) + (up2f + up3f)
    o10 = (cur[0] + cur[2]) + (lcur1 + lcur3)
    o11 = (cur[0] + cur[1]) + (cur[2] + cur[3])

    pr0 = pltpu.pack_elementwise([0.25 * o00, 0.25 * o01],
                                 packed_dtype=jnp.bfloat16)
    pr1 = pltpu.pack_elementwise([0.25 * o10, 0.25 * o11],
                                 packed_dtype=jnp.bfloat16)
    pieces = []
    for hh in range(th):
        pieces.append(pr0[:, w * hh:w * (hh + 1)])
        pieces.append(pr1[:, w * hh:w * (hh + 1)])
    o_ref[...] = jnp.concatenate(pieces, axis=1)


def kernel(x, weight, bias):
    n, cin, h, w = x.shape
    cout = weight.shape[0]
    cgrp = cout // 4

    th = 16 if (h % 16 == 0) else (8 if h % 8 == 0 else h)
    assert h % th == 0 and (th % 4 == 0 or h == th)

    xp = jnp.pad(x.astype(jnp.bfloat16), ((0, 0), (0, 0), (2, 2), (0, 0)))
    xp = xp.reshape(n, cin, (h + 4) * w)

    wp = weight.reshape(cgrp, 4, cin, 3, 3).transpose(1, 0, 3, 4, 2)
    wp = wp.reshape(cout, 9 * cin).astype(jnp.bfloat16)
    bp = bias.reshape(cgrp, 4).T.reshape(cout)
    bb = jnp.broadcast_to(bp[:, None], (cout, (th + 1) * w))

    body = functools.partial(_fused_kernel, th=th, w=w, cin=cin, cgrp=cgrp)
    flops = 2 * n * h * w * 9 * cin * cout * (th + 1) // th
    bytes_accessed = 4 * (xp.size + n * h * w * cout) + 2 * wp.size

    out = pl.pallas_call(
        body,
        out_shape=jax.ShapeDtypeStruct((n, cgrp, 2 * h * w), jnp.uint32),
        grid=(n, h // th),
        in_specs=[
            pl.BlockSpec((None, cin, th * w), lambda b, i: (b, 0, i)),
            pl.BlockSpec((None, cin, 4 * w),
                         lambda b, i: (b, 0, ((i + 1) * th) // 4)),
            pl.BlockSpec((cout, 9 * cin), lambda b, i: (0, 0)),
            pl.BlockSpec((cout, (th + 1) * w), lambda b, i: (0, 0)),
        ],
        out_specs=pl.BlockSpec((None, cgrp, 2 * th * w),
                               lambda b, i: (b, 0, i)),
        compiler_params=pltpu.CompilerParams(
            dimension_semantics=("parallel", "parallel"),
            vmem_limit_bytes=64 * 1024 * 1024),
        cost_estimate=pl.CostEstimate(flops=flops, transcendentals=0,
                                      bytes_accessed=bytes_accessed),
    )(xp, xp, wp, bb)
    bf = jax.lax.bitcast_convert_type(out, jnp.bfloat16)
    return bf.reshape(n, cgrp, 2 * h, 2 * w).astype(jnp.float32)


# speedup vs baseline: 1.0462x; 1.0462x over previous
import functools

import jax
import jax.numpy as jnp
from jax.experimental import pallas as pl
from jax.experimental.pallas import tpu as pltpu


def _fused_kernel(xm_ref, xh_ref, w_ref, b_ref, o_ref, *, th, w, cin, cgrp):
    lanes = (th + 1) * w
    xt = jnp.concatenate([xm_ref[...], xh_ref[...]], axis=-1)

    col = jax.lax.broadcasted_iota(jnp.int32, (cin, lanes), 1) % w
    zero = jnp.zeros((), xt.dtype)
    taps = []
    for dy in range(3):
        for dx in range(3):
            start = dy * w + dx - 1
            if start >= 0:
                t = xt[:, start:start + lanes]
            else:
                t = jnp.concatenate([xt[:, :1], xt[:, :lanes - 1]], axis=1)
            if dx == 0:
                t = jnp.where(col == 0, zero, t)
            elif dx == 2:
                t = jnp.where(col == w - 1, zero, t)
            taps.append(t)
    rhs = jnp.concatenate(taps, axis=0)

    acc = jnp.dot(w_ref[...], rhs,
                  preferred_element_type=jnp.float32) + b_ref[...]

    n_out = th * w
    cur = [acc[p * cgrp:(p + 1) * cgrp, w:] for p in range(4)]
    up2 = acc[2 * cgrp:3 * cgrp, :n_out]
    up3 = acc[3 * cgrp:4 * cgrp, :n_out]

    lane = jax.lax.broadcasted_iota(jnp.int32, (cgrp, n_out), 1)
    col0 = (lane % w) == 0
    row0 = jnp.logical_and(pl.program_id(1) == 0, lane < w)

    lcur1 = jnp.where(col0, cur[0], pltpu.roll(cur[1], 1, axis=1))
    lcur3 = jnp.where(col0, cur[2], pltpu.roll(cur[3], 1, axis=1))
    up2f = jnp.where(row0, cur[0], up2)
    up3f = jnp.where(row0, cur[1], up3)
    lup3 = jnp.where(row0, lcur1,
                     jnp.where(col0, up2, pltpu.roll(up3, 1, axis=1)))

    o00 = (cur[0] + lcur1) + (up2f + lup3)
    o01 = (cur[0] + cur[1]) + (up2f + up3f)
    o10 = (cur[0] + cur[2]) + (lcur1 + lcur3)
    o11 = (cur[0] + cur[1]) + (cur[2] + cur[3])

    o_ref[...] = (0.25 * jnp.concatenate([o00, o01, o10, o11], axis=0)
                  ).astype(o_ref.dtype)


def kernel(x, weight, bias):
    n, cin, h, w = x.shape
    cout = weight.shape[0]
    cgrp = cout // 4

    th = 16 if (h % 16 == 0) else (8 if h % 8 == 0 else h)
    assert h % th == 0 and (th % 4 == 0 or h == th)

    xp = jnp.pad(x.astype(jnp.bfloat16), ((0, 0), (0, 0), (2, 2), (0, 0)))
    xp = xp.reshape(n, cin, (h + 4) * w)

    wp = weight.reshape(cgrp, 4, cin, 3, 3).transpose(1, 0, 3, 4, 2)
    wp = wp.reshape(cout, 9 * cin).astype(jnp.bfloat16)
    bp = bias.reshape(cgrp, 4).T.reshape(cout)
    bb = jnp.broadcast_to(bp[:, None], (cout, (th + 1) * w))

    body = functools.partial(_fused_kernel, th=th, w=w, cin=cin, cgrp=cgrp)
    flops = 2 * n * h * w * 9 * cin * cout * (th + 1) // th
    bytes_accessed = 4 * (xp.size + n * h * w * cout) + 2 * wp.size

    out = pl.pallas_call(
        body,
        out_shape=jax.ShapeDtypeStruct((n, cout, h * w), jnp.bfloat16),
        grid=(n, h // th),
        in_specs=[
            pl.BlockSpec((None, cin, th * w), lambda b, i: (b, 0, i)),
            pl.BlockSpec((None, cin, 4 * w),
                         lambda b, i: (b, 0, ((i + 1) * th) // 4)),
            pl.BlockSpec((cout, 9 * cin), lambda b, i: (0, 0)),
            pl.BlockSpec((cout, (th + 1) * w), lambda b, i: (0, 0)),
        ],
        out_specs=pl.BlockSpec((None, cout, th * w), lambda b, i: (b, 0, i)),
        compiler_params=pltpu.CompilerParams(
            dimension_semantics=("parallel", "parallel"),
            vmem_limit_bytes=64 * 1024 * 1024),
        cost_estimate=pl.CostEstimate(flops=flops, transcendentals=0,
                                      bytes_accessed=bytes_accessed),
    )(xp, xp, wp, bb)

    out = out.reshape(n, 2, 2, cgrp, h, w).transpose(0, 3, 4, 1, 5, 2)
    return out.reshape(n, cgrp, 2 * h, 2 * w).astype(jnp.float32)
